# baseline (speedup 1.0000x reference)
import numpy as np
import jax
import jax.numpy as jnp
from jax import lax
from jax.experimental import pallas as pl
from jax.experimental.pallas import tpu as pltpu
from jax.experimental.pallas import tpu_sc as plsc

F = 26
D = 16
V = 100000
B = 4096
NPAIR = (F * (F - 1)) // 2
PAIR_COLS = NPAIR * 17
OWN0 = PAIR_COLS
LIN0 = OWN0 + F * D
OUT_COLS = LIN0 + F
G = 2
NW = 32
CHUNK = B // NW
NGRP = CHUNK // G
PAIRS = [(i, j) for i in range(F - 1) for j in range(i + 1, F)]

NQ = 4
EIDX = F * NQ
SROWS = NQ * V
VT_FULL = V // 128
VTAIL = V - VT_FULL * 128
NUNIT = NQ * VT_FULL
P1_ITER = (NUNIT + 3 * NW - 1) // (3 * NW)


def _p2_consts():
    n = G * EIDX
    ctab = np.zeros(n, np.int32)
    etab = np.zeros(n, np.int32)
    qtab = np.zeros(n, np.int32)
    for k in range(n):
        kk = k % EIDX
        etab[k] = k // EIDX
        ctab[k] = kk // NQ
        qtab[k] = (kk % NQ) * V
    return ctab, etab, qtab


_CTAB, _ETAB, _QTAB = _p2_consts()


def _srow(el, a, c):
    return el * EIDX + c * NQ + a // 8


def _scol(a):
    return (a % 8) * D


def _p1_issue(tp_hbm, lin_hbm, u, inbuf, linv, semi, seml):
    g = u // VT_FULL
    vt = u - g * VT_FULL
    v0 = vt * 128

    @pl.when(g < NQ - 1)
    def _full():
        for ai in range(8):
            pltpu.async_copy(
                tp_hbm.at[g * 8 + ai, pl.ds(0, 16), pl.ds(v0, 128)],
                inbuf.at[ai], semi)

    @pl.when(g == NQ - 1)
    def _last():
        for ai in range(2):
            pltpu.async_copy(
                tp_hbm.at[24 + ai, pl.ds(0, 16), pl.ds(v0, 128)],
                inbuf.at[ai], semi)
        pltpu.async_copy(lin_hbm.at[pl.ds(v0, 128)], linv, seml)


def _p1_work(tp_hbm, lin_hbm, scr_hbm, u, inbuf, linv, obuf, semi, seml, semo,
             first, iot):
    g = u // VT_FULL
    vt = u - g * VT_FULL
    v0 = vt * 128

    @pl.when(g < NQ - 1)
    def _wf():
        pltpu.make_async_copy(
            tp_hbm.at[pl.ds(0, 8), pl.ds(0, 16), pl.ds(0, 128)],
            inbuf, semi).wait()

    @pl.when(g == NQ - 1)
    def _wl():
        pltpu.make_async_copy(
            tp_hbm.at[pl.ds(0, 2), pl.ds(0, 16), pl.ds(0, 128)],
            inbuf.at[pl.ds(0, 2)], semi).wait()
        pltpu.make_async_copy(lin_hbm.at[pl.ds(0, 128)], linv, seml).wait()

    @pl.when(jnp.logical_not(first))
    def _wo():
        pltpu.make_async_copy(obuf, scr_hbm.at[pl.ds(0, 128 * 128)], semo).wait()

    aisel = [iot * 0 + ai for ai in range(8)]

    @pl.when(g < NQ - 1)
    def _tfull():
        def row4(r4, carry):
            vv0 = r4 * 4
            for dv in range(4):
                vvv = jnp.zeros((16,), jnp.int32) + (vv0 + dv)
                for ai in range(8):
                    val = plsc.load_gather(inbuf, [aisel[ai], iot, vvv])
                    obuf[pl.ds((vv0 + dv) * 128 + ai * 16, 16)] = val
            return carry
        lax.fori_loop(0, 32, row4, 0)

    @pl.when(g == NQ - 1)
    def _tlast():
        def row4(r4, carry):
            vv0 = r4 * 4
            for dv in range(4):
                vvv = jnp.zeros((16,), jnp.int32) + (vv0 + dv)
                for ai in range(2):
                    val = plsc.load_gather(inbuf, [aisel[ai], iot, vvv])
                    obuf[pl.ds((vv0 + dv) * 128 + ai * 16, 16)] = val
            return carry
        lax.fori_loop(0, 32, row4, 0)
        for r in range(8):
            lv = linv[pl.ds(r * 16, 16)]
            plsc.store_scatter(obuf, [(r * 16 + iot) * 128 + 32], lv)

    pltpu.async_copy(obuf, scr_hbm.at[pl.ds((g * V + v0) * 128, 128 * 128)],
                     semo)


def _p1_body(tp_hbm, lin_hbm, tail_hbm, scr_hbm,
             inA, inB, inC, linvA, linvB, linvC, obA, obB, obC,
             semiA, semiB, semiC, semlA, semlB, semlC, semoA, semoB, semoC):
    c = lax.axis_index("c")
    s = lax.axis_index("s")
    wid = s * 2 + c
    iot = lax.iota(jnp.int32, 16)

    _p1_issue(tp_hbm, lin_hbm, wid, inA, linvA, semiA, semlA)
    _p1_issue(tp_hbm, lin_hbm, wid + NW, inB, linvB, semiB, semlB)
    _p1_issue(tp_hbm, lin_hbm, wid + 2 * NW, inC, linvC, semiC, semlC)

    def p1_slot(t, off, inb, linv, ob, semi, seml, semo):
        u = wid + 3 * NW * t + off

        @pl.when(u < NUNIT)
        def _run():
            _p1_work(tp_hbm, lin_hbm, scr_hbm, u, inb, linv, ob,
                     semi, seml, semo, t == 0, iot)

            @pl.when(u + 3 * NW < NUNIT)
            def _next():
                _p1_issue(tp_hbm, lin_hbm, u + 3 * NW, inb, linv, semi, seml)

    def loop_body(t, carry):
        p1_slot(t, 0, inA, linvA, obA, semiA, semlA, semoA)
        p1_slot(t, NW, inB, linvB, obB, semiB, semlB, semoB)
        p1_slot(t, 2 * NW, inC, linvC, obC, semiC, semlC, semoC)
        return carry

    lax.fori_loop(0, P1_ITER, loop_body, 0)

    pltpu.make_async_copy(obA, scr_hbm.at[pl.ds(0, 128 * 128)], semoA).wait()
    pltpu.make_async_copy(obB, scr_hbm.at[pl.ds(0, 128 * 128)], semoB).wait()
    pltpu.make_async_copy(obC, scr_hbm.at[pl.ds(0, 128 * 128)], semoC).wait()

    @pl.when(wid < NQ)
    def _tail():
        g = wid
        n = VTAIL * 128
        pltpu.sync_copy(tail_hbm.at[pl.ds(g * n, n)], obA.at[pl.ds(0, n)])
        pltpu.sync_copy(obA.at[pl.ds(0, n)],
                        scr_hbm.at[pl.ds((g * V + VT_FULL * 128) * 128, n)])


def _compute_element(rowsv, el, orow, iot):
    for i in range(F):
        orow[pl.ds(OWN0 + i * D, D)] = rowsv[_srow(el, i, i), pl.ds(_scol(i), D)]
    for p, (i, j) in enumerate(PAIRS):
        h = (rowsv[_srow(el, j, i), pl.ds(_scol(j), D)]
             * rowsv[_srow(el, i, j), pl.ds(_scol(i), D)])
        orow[pl.ds(17 * p, 16)] = h
    def inner_group(t, carry):
        p0 = t * 16
        cnt = NPAIR - p0
        msk = iot < cnt
        colv = jnp.where(msk, 17 * p0 + 17 * iot, 0)
        acc = jnp.zeros((16,), jnp.float32)
        for dd in range(16):
            acc = acc + plsc.load_gather(orow, [colv + dd])
        plsc.store_scatter(orow, [colv + 16], acc, mask=msk)
        return carry

    lax.fori_loop(0, (NPAIR + 15) // 16, inner_group, 0)
    rbase = el * EIDX + 3
    lv0 = plsc.load_gather(rowsv, [rbase + NQ * iot, jnp.zeros((16,), jnp.int32) + 32])
    orow[pl.ds(LIN0, 16)] = lv0
    r2 = jnp.where(iot < F - 16, rbase + NQ * (16 + iot), rbase)
    lv1 = plsc.load_gather(rowsv, [r2, jnp.zeros((16,), jnp.int32) + 32])
    plsc.store_scatter(orow, [LIN0 + 16 + iot], lv1, mask=iot < F - 16)


def _build_idx(idxref, xv, ctabv, etabv, qtabv, e0):
    for r in range(G * EIDX // 16):
        sl = pl.ds(r * 16, 16)
        xval = plsc.load_gather(xv, [e0 + etabv[sl], ctabv[sl]])
        idxref[sl] = xval + qtabv[sl]


def _p2_body(x_hbm, scr_hbm, ctab_hbm, etab_hbm, qtab_hbm, out_hbm,
             xv, ctabv, etabv, qtabv, idxA, idxB, rowsA, rowsB,
             obufA, obufB,
             semgA, semgB, semwA, semwB):
    c = lax.axis_index("c")
    s = lax.axis_index("s")
    wid = s * 2 + c
    base = wid * CHUNK

    pltpu.sync_copy(x_hbm.at[pl.ds(base, CHUNK)], xv)
    pltpu.sync_copy(ctab_hbm, ctabv)
    pltpu.sync_copy(etab_hbm, etabv)
    pltpu.sync_copy(qtab_hbm, qtabv)

    iot = lax.iota(jnp.int32, 16)

    _build_idx(idxA, xv, ctabv, etabv, qtabv, 0)
    pltpu.async_copy(scr_hbm.at[idxA], rowsA, semgA)
    _build_idx(idxB, xv, ctabv, etabv, qtabv, G)
    pltpu.async_copy(scr_hbm.at[idxB], rowsB, semgB)

    def slot(m, slot_id, idxref, rowsref, obuf, semg, semw):
        g = 2 * m + slot_id
        e0 = g * G
        pltpu.make_async_copy(scr_hbm.at[idxref], rowsref, semg).wait()

        @pl.when(m > 0)
        def _wait_prev_write():
            pltpu.make_async_copy(obuf, out_hbm.at[pl.ds(base, G)], semw).wait()

        for el in range(G):
            _compute_element(rowsref, el, obuf.at[el], iot)
        pltpu.async_copy(obuf, out_hbm.at[pl.ds(base + e0, G)], semw)

        @pl.when(m < NGRP // 2 - 1)
        def _prefetch_next():
            _build_idx(idxref, xv, ctabv, etabv, qtabv, e0 + 2 * G)
            pltpu.async_copy(scr_hbm.at[idxref], rowsref, semg)

    def loop_body(m, carry):
        slot(m, 0, idxA, rowsA, obufA, semgA, semwA)
        slot(m, 1, idxB, rowsB, obufB, semgB, semwB)
        return carry

    lax.fori_loop(0, NGRP // 2, loop_body, 0)

    pltpu.make_async_copy(obufA, out_hbm.at[pl.ds(base, G)], semwA).wait()
    pltpu.make_async_copy(obufB, out_hbm.at[pl.ds(base, G)], semwB).wait()


def kernel(x, tables, linear_w):
    mesh = plsc.VectorSubcoreMesh(core_axis_name="c", subcore_axis_name="s")

    p1 = pl.kernel(
        _p1_body,
        out_type=jax.ShapeDtypeStruct((SROWS * 128,), jnp.float32),
        mesh=mesh,
        scratch_types=[
            pltpu.VMEM((8, 16, 128), jnp.float32),
            pltpu.VMEM((8, 16, 128), jnp.float32),
            pltpu.VMEM((8, 16, 128), jnp.float32),
            pltpu.VMEM((128,), jnp.float32),
            pltpu.VMEM((128,), jnp.float32),
            pltpu.VMEM((128,), jnp.float32),
            pltpu.VMEM((128 * 128,), jnp.float32),
            pltpu.VMEM((128 * 128,), jnp.float32),
            pltpu.VMEM((128 * 128,), jnp.float32),
            pltpu.SemaphoreType.DMA,
            pltpu.SemaphoreType.DMA,
            pltpu.SemaphoreType.DMA,
            pltpu.SemaphoreType.DMA,
            pltpu.SemaphoreType.DMA,
            pltpu.SemaphoreType.DMA,
            pltpu.SemaphoreType.DMA,
            pltpu.SemaphoreType.DMA,
            pltpu.SemaphoreType.DMA,
        ],
        compiler_params=pltpu.CompilerParams(
            use_tc_tiling_on_sc=True, needs_layout_passes=False
        ),
    )

    p2 = pl.kernel(
        _p2_body,
        out_type=jax.ShapeDtypeStruct((B, OUT_COLS), jnp.float32),
        mesh=mesh,
        scratch_types=[
            pltpu.VMEM((CHUNK, F), jnp.int32),
            pltpu.VMEM((G * EIDX,), jnp.int32),
            pltpu.VMEM((G * EIDX,), jnp.int32),
            pltpu.VMEM((G * EIDX,), jnp.int32),
            pltpu.VMEM((G * EIDX,), jnp.int32),
            pltpu.VMEM((G * EIDX,), jnp.int32),
            pltpu.VMEM((G * EIDX, 128), jnp.float32),
            pltpu.VMEM((G * EIDX, 128), jnp.float32),
            pltpu.VMEM((G, OUT_COLS), jnp.float32),
            pltpu.VMEM((G, OUT_COLS), jnp.float32),
            pltpu.SemaphoreType.DMA,
            pltpu.SemaphoreType.DMA,
            pltpu.SemaphoreType.DMA,
            pltpu.SemaphoreType.DMA,
        ],
        compiler_params=pltpu.CompilerParams(
            use_tc_tiling_on_sc=False, needs_layout_passes=False
        ),
    )

    tp = jnp.transpose(tables, (0, 2, 1))
    tails = jnp.transpose(tables[:, VT_FULL * 128:, :], (1, 0, 2))
    tails = jnp.pad(tails, ((0, 0), (0, 32 - F), (0, 0)))
    tails = jnp.transpose(tails.reshape(VTAIL, NQ, 128), (1, 0, 2))
    tails = tails.at[NQ - 1, :, 32].set(linear_w[VT_FULL * 128:, 0])
    scr = p1(tp, linear_w.reshape(-1), tails.reshape(-1))
    scr2d = scr.reshape(SROWS, 128)
    ctab = jnp.asarray(_CTAB)
    etab = jnp.asarray(_ETAB)
    qtab = jnp.asarray(_QTAB)
    return p2(x.astype(jnp.int32), scr2d, ctab, etab, qtab)
